# Initial kernel scaffold; baseline (speedup 1.0000x reference)
import jax
import jax.numpy as jnp
from jax.experimental import pallas as pl


def kernel(inputs):
    raise NotImplementedError("write your pallas kernel here")


# speedup vs baseline: 197.5365x; 197.5365x over previous
import jax
import jax.numpy as jnp
from jax.experimental import pallas as pl
from jax.experimental.pallas import tpu as pltpu

_K = 50
_N = 200
_W = 64


def _kmax_block(x_ref, o_ref):
    x = x_ref[...]
    g2 = x.shape[0]
    g = g2 // 2
    x = x.reshape(g, 2, _N, _W)
    y = jnp.concatenate([x[:, 0], x[:, 1]], axis=2)
    iota = jax.lax.broadcasted_iota(jnp.int32, y.shape, 1)
    neg = jnp.float32(-jnp.inf)
    outs = []
    for i in range(_K):
        m = jnp.max(y, axis=1, keepdims=True)
        outs.append(m)
        if i < _K - 1:
            idx = jnp.min(jnp.where(y == m, iota, _N), axis=1, keepdims=True)
            y = jnp.where(iota == idx, neg, y)
    s = jnp.concatenate(outs, axis=1)
    s = jnp.stack([s[:, :, :_W], s[:, :, _W:]], axis=1)
    o_ref[...] = s.reshape(g2, _K, _W)


def kernel(inputs):
    b = inputs.shape[0]
    x = inputs.reshape(b, _N, _W)
    g2 = 8
    out = pl.pallas_call(
        _kmax_block,
        grid=(b // g2,),
        in_specs=[pl.BlockSpec((g2, _N, _W), lambda i: (i, 0, 0))],
        out_specs=pl.BlockSpec((g2, _K, _W), lambda i: (i, 0, 0)),
        out_shape=jax.ShapeDtypeStruct((b, _K, _W), jnp.float32),
        compiler_params=pltpu.CompilerParams(
            dimension_semantics=("arbitrary",),
        ),
    )(x)
    return out.reshape(b, _K, _W, 1)


# speedup vs baseline: 197.5835x; 1.0002x over previous
import jax
import jax.numpy as jnp
from jax.experimental import pallas as pl
from jax.experimental.pallas import tpu as pltpu

_K = 50
_N = 200
_W = 64


def _kmax_block(x_ref, o_ref):
    x = x_ref[...]
    g2 = x.shape[0]
    g = g2 // 2
    x = x.reshape(g, 2, _N, _W)
    y = jnp.concatenate([x[:, 0], x[:, 1]], axis=2)
    iota = jax.lax.broadcasted_iota(jnp.int32, y.shape, 1)
    neg = jnp.float32(-jnp.inf)
    outs = []
    for i in range(_K):
        m = jnp.max(y, axis=1, keepdims=True)
        outs.append(m)
        if i < _K - 1:
            idx = jnp.min(jnp.where(y == m, iota, _N), axis=1, keepdims=True)
            y = jnp.where(iota == idx, neg, y)
    s = jnp.concatenate(outs, axis=1)
    s = jnp.stack([s[:, :, :_W], s[:, :, _W:]], axis=1)
    o_ref[...] = s.reshape(g2, _K, _W)


def kernel(inputs):
    b = inputs.shape[0]
    x = inputs.reshape(b, _N, _W)
    g2 = 8
    out = pl.pallas_call(
        _kmax_block,
        grid=(b // g2,),
        in_specs=[pl.BlockSpec((g2, _N, _W), lambda i: (i, 0, 0))],
        out_specs=pl.BlockSpec((g2, _K, _W), lambda i: (i, 0, 0)),
        out_shape=jax.ShapeDtypeStruct((b, _K, _W), jnp.float32),
        compiler_params=pltpu.CompilerParams(
            dimension_semantics=("parallel",),
        ),
    )(x)
    return out.reshape(b, _K, _W, 1)


# speedup vs baseline: 207.9207x; 1.0523x over previous
import jax
import jax.numpy as jnp
from jax.experimental import pallas as pl
from jax.experimental.pallas import tpu as pltpu

_K = 50
_N = 200
_W = 64


def _kmax_block(x_ref, o_ref):
    x = x_ref[...]
    g2 = x.shape[0]
    g = g2 // 2
    x = x.reshape(g, 2, _N, _W)
    y = jnp.concatenate([x[:, 0], x[:, 1]], axis=2)
    iota = jax.lax.broadcasted_iota(jnp.int32, y.shape, 1).astype(jnp.float32)
    neg = jnp.float32(-jnp.inf)
    big = jnp.float32(_N)
    outs = []
    for i in range(_K):
        m = jnp.max(y, axis=1, keepdims=True)
        outs.append(m)
        if i < _K - 1:
            idx = jnp.min(jnp.where(y == m, iota, big), axis=1, keepdims=True)
            y = jnp.where(iota == idx, neg, y)
    s = jnp.concatenate(outs, axis=1)
    s = jnp.stack([s[:, :, :_W], s[:, :, _W:]], axis=1)
    o_ref[...] = s.reshape(g2, _K, _W)


def kernel(inputs):
    b = inputs.shape[0]
    x = inputs.reshape(b, _N, _W)
    g2 = 8
    out = pl.pallas_call(
        _kmax_block,
        grid=(b // g2,),
        in_specs=[pl.BlockSpec((g2, _N, _W), lambda i: (i, 0, 0))],
        out_specs=pl.BlockSpec((g2, _K, _W), lambda i: (i, 0, 0)),
        out_shape=jax.ShapeDtypeStruct((b, _K, _W), jnp.float32),
        compiler_params=pltpu.CompilerParams(
            dimension_semantics=("parallel",),
        ),
    )(x)
    return out.reshape(b, _K, _W, 1)


# speedup vs baseline: 278.0288x; 1.3372x over previous
import functools

import jax
import jax.numpy as jnp
from jax import lax
from jax.experimental import pallas as pl
from jax.experimental.pallas import tpu as pltpu
from jax.experimental.pallas import tpu_sc as plsc

_K = 50
_N = 200
_W = 64
_B = 1024
_NW = 32
_SLABS = _B // _NW
_NG = _N // 8
_MINKEY = -2147483648


def _treemax(vs):
    vs = list(vs)
    while len(vs) > 1:
        nxt = [jnp.maximum(vs[i], vs[i + 1]) for i in range(0, len(vs) - 1, 2)]
        if len(vs) % 2:
            nxt.append(vs[-1])
        vs = nxt
    return vs[0]


def _sc_body(x_hbm, o_hbm, xv, kv, ov, gv):
    wid = lax.axis_index("s") * 2 + lax.axis_index("c")
    iota = lax.iota(jnp.int32, 16)
    minkey = jnp.full((16,), _MINKEY, jnp.int32)

    def slab(s, _):
        boff = (wid * _SLABS + s) * (_N * _W)
        pltpu.sync_copy(x_hbm.at[pl.ds(boff, _N * _W)], xv)

        def colgroup(cg, _):
            cbase = cg * 16
            colv = cbase + iota

            def build(g, _):
                ks = []
                for j in range(8):
                    off = g * (8 * _W) + j * _W + cbase
                    v = xv[pl.ds(off, 16)]
                    b = lax.bitcast_convert_type(v, jnp.int32)
                    t = b ^ (jnp.right_shift(b, 31) & 0x7FFFFFFF)
                    key = (t & -256) | (255 - (g * 8 + j))
                    kv[pl.ds(off, 16)] = key
                    ks.append(key)
                gv[pl.ds(g * 16, 16)] = _treemax(ks)
                return _

            lax.fori_loop(0, _NG, build, None)

            def extract(i, _):
                gs = [gv[pl.ds(16 * g, 16)] for g in range(_NG)]
                m = _treemax(gs)
                nn = 255 - (m & 255)
                fi = nn * _W + colv
                val = plsc.load_gather(xv, [fi])
                ov[pl.ds(i * _W + cbase, 16)] = val
                plsc.store_scatter(kv, [fi], minkey)
                g = jnp.right_shift(nn, 3)
                b0 = g * (8 * _W) + colv
                ks = [plsc.load_gather(kv, [b0 + j * _W]) for j in range(8)]
                plsc.store_scatter(gv, [g * 16 + iota], _treemax(ks))
                return _

            lax.fori_loop(0, _K, extract, None)
            return _

        lax.fori_loop(0, _W // 16, colgroup, None)
        pltpu.sync_copy(ov, o_hbm.at[pl.ds((wid * _SLABS + s) * (_K * _W), _K * _W)])
        return _

    lax.fori_loop(0, _SLABS, slab, None)


_sc_topk = functools.partial(
    pl.kernel,
    out_type=jax.ShapeDtypeStruct((_B * _K * _W,), jnp.float32),
    mesh=plsc.VectorSubcoreMesh(core_axis_name="c", subcore_axis_name="s"),
    compiler_params=pltpu.CompilerParams(needs_layout_passes=False),
    scratch_types=[
        pltpu.VMEM((_N * _W,), jnp.float32),
        pltpu.VMEM((_N * _W,), jnp.int32),
        pltpu.VMEM((_K * _W,), jnp.float32),
        pltpu.VMEM((_NG * 16,), jnp.int32),
    ],
)(_sc_body)


def _tc_block(x_ref, o_ref):
    x = x_ref[...]
    g2 = x.shape[0]
    g = g2 // 2
    x = x.reshape(g, 2, _N, _W)
    y = jnp.concatenate([x[:, 0], x[:, 1]], axis=2)
    iota = jax.lax.broadcasted_iota(jnp.int32, y.shape, 1).astype(jnp.float32)
    neg = jnp.float32(-jnp.inf)
    big = jnp.float32(_N)
    outs = []
    for i in range(_K):
        m = jnp.max(y, axis=1, keepdims=True)
        outs.append(m)
        if i < _K - 1:
            idx = jnp.min(jnp.where(y == m, iota, big), axis=1, keepdims=True)
            y = jnp.where(iota == idx, neg, y)
    s = jnp.concatenate(outs, axis=1)
    s = jnp.stack([s[:, :, :_W], s[:, :, _W:]], axis=1)
    o_ref[...] = s.reshape(g2, _K, _W)


def _tc_topk(x):
    b = x.shape[0]
    g2 = 8
    return pl.pallas_call(
        _tc_block,
        grid=(b // g2,),
        in_specs=[pl.BlockSpec((g2, _N, _W), lambda i: (i, 0, 0))],
        out_specs=pl.BlockSpec((g2, _K, _W), lambda i: (i, 0, 0)),
        out_shape=jax.ShapeDtypeStruct((b, _K, _W), jnp.float32),
        compiler_params=pltpu.CompilerParams(
            dimension_semantics=("arbitrary",),
        ),
    )(x)


def kernel(inputs):
    out = _sc_topk(inputs.reshape(-1))
    return out.reshape(_B, _K, _W, 1)
